# baseline (speedup 1.0000x reference)
import numpy as np

import jax
import jax.numpy as jnp
from jax import lax
from jax.experimental import pallas as pl
from jax.experimental.pallas import tpu as pltpu
from jax.experimental.pallas import tpu_sc as plsc

_S = 32
_T = 32
_G = 16
_RADIUS = 1.1
_SCALE = 100.0
_N = 50000

_a64 = _SCALE * np.linspace(-_RADIUS, _RADIUS, _S)
_H_STEP = float(_a64[1] - _a64[0])
_A0 = float(_a64[0])
_INV_H = float(1.0 / _H_STEP)
_EXP_KH = {-1: float(np.exp(_H_STEP)), 1: float(np.exp(-_H_STEP))}

_TC_CHUNK = 2048
_N_TC = 0 * _TC_CHUNK
_NC = 2
_NS = 16
_W = 1568
_N_SC_PAD = _NC * _NS * _W
_NPAD = _N_TC + _N_SC_PAD

_HIST = _G * 33 * _T
_CORR = _G * _S * _T
_ACC = _HIST + _CORR
_STRIPE = _ACC // _NS

_H = _S // 2
_C0 = float((_a64[0] + _a64[_H - 1]) / 2.0)
_C1 = float((_a64[_H] + _a64[-1]) / 2.0)
_KCONST = [float(np.exp((_C0 if s < _H else _C1) - _a64[s]))
           for s in range(_S)]


def _sc_body(x0_hbm, x1_hbm, x2_hbm, b_hbm, v_hbm, out_hbm,
             x0_v, x1_v, x2_v, b_v, v_v, acc_v, tmp_v, red_v,
             histg_v, corrg_v, outg_v, slots_sh, red_sh):
    c = lax.axis_index("c")
    sid = lax.axis_index("s")
    wid = c * _NS + sid
    base = _N_TC + wid * _W
    pltpu.sync_copy(x0_hbm.at[pl.ds(base, _W)], x0_v.at[pl.ds(0, _W)])
    pltpu.sync_copy(x1_hbm.at[pl.ds(base, _W)], x1_v.at[pl.ds(0, _W)])
    pltpu.sync_copy(x2_hbm.at[pl.ds(base, _W)], x2_v.at[pl.ds(0, _W)])
    pltpu.sync_copy(b_hbm.at[pl.ds(base, _W)], b_v.at[pl.ds(0, _W)])
    pltpu.sync_copy(v_hbm, v_v)

    zero16 = jnp.zeros((16,), jnp.float32)

    def zbody(i, carry):
        acc_v[pl.ds(i * 16, 16)] = zero16
        return carry

    lax.fori_loop(0, _ACC // 16, zbody, 0)

    vrows = [[v_v[d, pl.ds(hh * 16, 16)] for hh in range(2)] for d in range(3)]
    t_idx = [lax.iota(jnp.int32, 16) + hh * 16 for hh in range(2)]

    def body(n, carry):
        x0 = x0_v[pl.ds(n, 16)][0]
        x1 = x1_v[pl.ds(n, 16)][0]
        x2 = x2_v[pl.ds(n, 16)][0]
        g = b_v[pl.ds(n, 16)][0]
        goff_h = g * (33 * _T)
        goff_c = _HIST + g * (_S * _T)
        valid = jnp.where(base + n < _N, 1.0, 0.0).astype(jnp.float32)
        vvec = valid + zero16
        for hh in range(2):
            bvec = x0 * vrows[0][hh] + x1 * vrows[1][hh] + x2 * vrows[2][hh]
            u = (bvec - _A0) * _INV_H
            bini = jnp.clip(u + 1.0, 0.0, 32.9).astype(jnp.int32)
            plsc.addupdate_scatter(acc_v, [goff_h + bini * _T + t_idx[hh]], vvec)
            s0 = jnp.clip(u + 0.5, 0.0, 31.9).astype(jnp.int32)
            s0f = s0.astype(jnp.float32)
            em = jnp.exp((u - s0f) * _H_STEP)
            for k in (-1, 0, 1):
                e_k = em * _EXP_KH[k] if k else em
                sig = 1.0 / (1.0 + e_k)
                skf = s0f + k
                stepv = jnp.where(u < skf, 1.0, 0.0)
                cval = (sig - stepv) * vvec
                if k == -1:
                    cval = jnp.where(s0 >= 1, cval, 0.0)
                    sk = jnp.maximum(s0 + k, 0)
                elif k == 1:
                    cval = jnp.where(s0 <= 30, cval, 0.0)
                    sk = jnp.minimum(s0 + k, 31)
                else:
                    sk = s0
                plsc.addupdate_scatter(acc_v, [goff_c + sk * _T + t_idx[hh]], cval)
        return carry

    def body4(i, carry):
        for j in range(4):
            body(i * 4 + j, carry)
        return carry

    lax.fori_loop(0, _W // 4, body4, 0)

    pltpu.sync_copy(acc_v, slots_sh.at[pl.ds(sid * _ACC, _ACC)])
    plsc.subcore_barrier()

    soff = sid * _STRIPE
    for j in range(_NS):
        pltpu.sync_copy(slots_sh.at[pl.ds(j * _ACC + soff, _STRIPE)],
                        tmp_v.at[pl.ds(j * _STRIPE, _STRIPE)])

    def rbody(i, carry):
        acc = tmp_v[pl.ds(i * 16, 16)]
        for j in range(1, _NS):
            acc = acc + tmp_v[pl.ds(j * _STRIPE + i * 16, 16)]
        red_v[pl.ds(i * 16, 16)] = acc
        return carry

    lax.fori_loop(0, _STRIPE // 16, rbody, 0)
    pltpu.sync_copy(red_v, red_sh.at[pl.ds(soff, _STRIPE)])
    plsc.subcore_barrier()

    g = sid
    pltpu.sync_copy(red_sh.at[pl.ds(g * (33 * _T), 33 * _T)], histg_v)
    pltpu.sync_copy(red_sh.at[pl.ds(_HIST + g * (_S * _T), _S * _T)], corrg_v)
    run = [zero16, zero16]
    for s in range(_S):
        for hh in range(2):
            off = s * _T + hh * 16
            run[hh] = run[hh] + histg_v[pl.ds(off, 16)]
            outg_v[pl.ds(off, 16)] = run[hh] + corrg_v[pl.ds(off, 16)]
    pltpu.sync_copy(outg_v, out_hbm.at[c, g])


def _run_sc(x0, x1, x2, bp, vs):
    mesh = plsc.VectorSubcoreMesh(core_axis_name="c", subcore_axis_name="s")
    run = pl.kernel(
        _sc_body,
        out_type=jax.ShapeDtypeStruct((_NC, _G, _S * _T), jnp.float32),
        mesh=mesh,
        compiler_params=pltpu.CompilerParams(
            needs_layout_passes=False, use_tc_tiling_on_sc=False),
        scratch_types=[
            pltpu.VMEM((_W + 16,), jnp.float32),
            pltpu.VMEM((_W + 16,), jnp.float32),
            pltpu.VMEM((_W + 16,), jnp.float32),
            pltpu.VMEM((_W + 16,), jnp.int32),
            pltpu.VMEM((3, _T), jnp.float32),
            pltpu.VMEM((_ACC,), jnp.float32),
            pltpu.VMEM((_NS * _STRIPE,), jnp.float32),
            pltpu.VMEM((_STRIPE,), jnp.float32),
            pltpu.VMEM((33 * _T,), jnp.float32),
            pltpu.VMEM((_S * _T,), jnp.float32),
            pltpu.VMEM((_S * _T,), jnp.float32),
            pltpu.VMEM_SHARED((_NS * _ACC,), jnp.float32),
            pltpu.VMEM_SHARED((_ACC,), jnp.float32),
        ],
    )
    return run(x0, x1, x2, bp, vs)


def _tc_kernel(xt_ref, b_ref, vt_ref, out_ref):
    i = pl.program_id(0)

    @pl.when(i == 0)
    def _init():
        out_ref[...] = jnp.zeros_like(out_ref)

    nh = jnp.dot(vt_ref[...], xt_ref[...],
                 preferred_element_type=jnp.float32)
    e0 = jnp.exp(jnp.clip(nh - _C0, -88.0, 88.0)).astype(jnp.bfloat16)
    e1 = jnp.exp(jnp.clip(nh - _C1, -88.0, 88.0)).astype(jnp.bfloat16)
    one = jnp.bfloat16(1.0)
    blocks = []
    for s in range(_S):
        e = e0 if s < _H else e1
        m1 = one + e * jnp.bfloat16(_KCONST[s])
        blocks.append(one / m1)
    sig = jnp.concatenate(blocks, axis=0)
    seg = b_ref[...]
    oh = (seg == lax.broadcasted_iota(jnp.int32, (1, _G), 1)
          ).astype(jnp.bfloat16)
    out_ref[...] += jnp.dot(sig, oh,
                            preferred_element_type=jnp.float32)


def _run_tc(xt, batch2d, vt):
    st = _S * _T
    grid = _N_TC // _TC_CHUNK
    return pl.pallas_call(
        _tc_kernel,
        grid=(grid,),
        in_specs=[
            pl.BlockSpec((3, _TC_CHUNK), lambda i: (0, i)),
            pl.BlockSpec((_TC_CHUNK, 1), lambda i: (i, 0)),
            pl.BlockSpec((_T, 3), lambda i: (0, 0)),
        ],
        out_specs=pl.BlockSpec((st, _G), lambda i: (0, 0)),
        out_shape=jax.ShapeDtypeStruct((st, _G), jnp.float32),
    )(xt, batch2d, vt)


@jax.jit
def kernel(x, batch, v):
    xp = jnp.pad(x, ((0, _NPAD - _N), (0, 0)))
    bp = jnp.pad(batch, (0, _NPAD - _N)).astype(jnp.int32)
    vs = (_SCALE * v).astype(jnp.float32)
    x0, x1, x2 = xp[:, 0], xp[:, 1], xp[:, 2]
    out_sc = _run_sc(x0, x1, x2, bp, vs)
    out = out_sc.sum(axis=0).reshape(_G, _S, _T)
    if _N_TC:
        xt = xp[:_N_TC].T
        batch2d = bp[:_N_TC].reshape(_N_TC, 1)
        out_tc = _run_tc(xt, batch2d, vs.T)
        out = out + out_tc.T.reshape(_G, _S, _T)
    return out


# speedup vs baseline: 1.6773x; 1.6773x over previous
import numpy as np

import jax
import jax.numpy as jnp
from jax import lax
from jax.experimental import pallas as pl
from jax.experimental.pallas import tpu as pltpu
from jax.experimental.pallas import tpu_sc as plsc

_S = 32
_T = 32
_G = 16
_RADIUS = 1.1
_SCALE = 100.0
_N = 50000

_a64 = _SCALE * np.linspace(-_RADIUS, _RADIUS, _S)
_H_STEP = float(_a64[1] - _a64[0])
_A0 = float(_a64[0])
_INV_H = float(1.0 / _H_STEP)
_EXP_KH = {-1: float(np.exp(_H_STEP)), 1: float(np.exp(-_H_STEP))}

_TC_CHUNK = 2048
_N_TC = 16 * _TC_CHUNK
_NC = 2
_NS = 16
_W = 544
_N_SC_PAD = _NC * _NS * _W
_NPAD = _N_TC + _N_SC_PAD

_HIST = _G * 33 * _T
_CORR = _G * _S * _T
_ACC = _HIST + _CORR
_STRIPE = _ACC // _NS

_H = _S // 2
_C0 = float((_a64[0] + _a64[_H - 1]) / 2.0)
_C1 = float((_a64[_H] + _a64[-1]) / 2.0)
_KCONST = [float(np.exp((_C0 if s < _H else _C1) - _a64[s]))
           for s in range(_S)]


def _sc_body(x0_hbm, x1_hbm, x2_hbm, b_hbm, v_hbm, out_hbm,
             x0_v, x1_v, x2_v, b_v, v_v, acc_v, tmp_v, red_v,
             histg_v, corrg_v, outg_v, slots_sh, red_sh):
    c = lax.axis_index("c")
    sid = lax.axis_index("s")
    wid = c * _NS + sid
    base = _N_TC + wid * _W
    pltpu.sync_copy(x0_hbm.at[pl.ds(base, _W)], x0_v.at[pl.ds(0, _W)])
    pltpu.sync_copy(x1_hbm.at[pl.ds(base, _W)], x1_v.at[pl.ds(0, _W)])
    pltpu.sync_copy(x2_hbm.at[pl.ds(base, _W)], x2_v.at[pl.ds(0, _W)])
    pltpu.sync_copy(b_hbm.at[pl.ds(base, _W)], b_v.at[pl.ds(0, _W)])
    pltpu.sync_copy(v_hbm, v_v)

    zero16 = jnp.zeros((16,), jnp.float32)

    def zbody(i, carry):
        acc_v[pl.ds(i * 16, 16)] = zero16
        return carry

    lax.fori_loop(0, _ACC // 16, zbody, 0)

    vrows = [[v_v[d, pl.ds(hh * 16, 16)] for hh in range(2)] for d in range(3)]
    t_idx = [lax.iota(jnp.int32, 16) + hh * 16 for hh in range(2)]

    def body(n, carry):
        x0 = x0_v[pl.ds(n, 16)][0]
        x1 = x1_v[pl.ds(n, 16)][0]
        x2 = x2_v[pl.ds(n, 16)][0]
        g = b_v[pl.ds(n, 16)][0]
        goff_h = g * (33 * _T)
        goff_c = _HIST + g * (_S * _T)
        valid = jnp.where(base + n < _N, 1.0, 0.0).astype(jnp.float32)
        vvec = valid + zero16
        for hh in range(2):
            bvec = x0 * vrows[0][hh] + x1 * vrows[1][hh] + x2 * vrows[2][hh]
            u = (bvec - _A0) * _INV_H
            bini = jnp.clip(u + 1.0, 0.0, 32.9).astype(jnp.int32)
            plsc.addupdate_scatter(acc_v, [goff_h + bini * _T + t_idx[hh]], vvec)
            s0 = jnp.clip(u + 0.5, 0.0, 31.9).astype(jnp.int32)
            s0f = s0.astype(jnp.float32)
            em = jnp.exp((u - s0f) * _H_STEP)
            for k in (-1, 0, 1):
                e_k = em * _EXP_KH[k] if k else em
                sig = 1.0 / (1.0 + e_k)
                skf = s0f + k
                stepv = jnp.where(u < skf, 1.0, 0.0)
                cval = (sig - stepv) * vvec
                if k == -1:
                    cval = jnp.where(s0 >= 1, cval, 0.0)
                    sk = jnp.maximum(s0 + k, 0)
                elif k == 1:
                    cval = jnp.where(s0 <= 30, cval, 0.0)
                    sk = jnp.minimum(s0 + k, 31)
                else:
                    sk = s0
                plsc.addupdate_scatter(acc_v, [goff_c + sk * _T + t_idx[hh]], cval)
        return carry

    lax.fori_loop(0, _W, body, 0)

    pltpu.sync_copy(acc_v, slots_sh.at[pl.ds(sid * _ACC, _ACC)])
    plsc.subcore_barrier()

    soff = sid * _STRIPE
    for j in range(_NS):
        pltpu.sync_copy(slots_sh.at[pl.ds(j * _ACC + soff, _STRIPE)],
                        tmp_v.at[pl.ds(j * _STRIPE, _STRIPE)])

    def rbody(i, carry):
        acc = tmp_v[pl.ds(i * 16, 16)]
        for j in range(1, _NS):
            acc = acc + tmp_v[pl.ds(j * _STRIPE + i * 16, 16)]
        red_v[pl.ds(i * 16, 16)] = acc
        return carry

    lax.fori_loop(0, _STRIPE // 16, rbody, 0)
    pltpu.sync_copy(red_v, red_sh.at[pl.ds(soff, _STRIPE)])
    plsc.subcore_barrier()

    g = sid
    pltpu.sync_copy(red_sh.at[pl.ds(g * (33 * _T), 33 * _T)], histg_v)
    pltpu.sync_copy(red_sh.at[pl.ds(_HIST + g * (_S * _T), _S * _T)], corrg_v)
    run = [zero16, zero16]
    for s in range(_S):
        for hh in range(2):
            off = s * _T + hh * 16
            run[hh] = run[hh] + histg_v[pl.ds(off, 16)]
            outg_v[pl.ds(off, 16)] = run[hh] + corrg_v[pl.ds(off, 16)]
    pltpu.sync_copy(outg_v, out_hbm.at[c, g])


def _run_sc(x0, x1, x2, bp, vs):
    mesh = plsc.VectorSubcoreMesh(core_axis_name="c", subcore_axis_name="s")
    run = pl.kernel(
        _sc_body,
        out_type=jax.ShapeDtypeStruct((_NC, _G, _S * _T), jnp.float32),
        mesh=mesh,
        compiler_params=pltpu.CompilerParams(
            needs_layout_passes=False, use_tc_tiling_on_sc=False),
        scratch_types=[
            pltpu.VMEM((_W + 16,), jnp.float32),
            pltpu.VMEM((_W + 16,), jnp.float32),
            pltpu.VMEM((_W + 16,), jnp.float32),
            pltpu.VMEM((_W + 16,), jnp.int32),
            pltpu.VMEM((3, _T), jnp.float32),
            pltpu.VMEM((_ACC,), jnp.float32),
            pltpu.VMEM((_NS * _STRIPE,), jnp.float32),
            pltpu.VMEM((_STRIPE,), jnp.float32),
            pltpu.VMEM((33 * _T,), jnp.float32),
            pltpu.VMEM((_S * _T,), jnp.float32),
            pltpu.VMEM((_S * _T,), jnp.float32),
            pltpu.VMEM_SHARED((_NS * _ACC,), jnp.float32),
            pltpu.VMEM_SHARED((_ACC,), jnp.float32),
        ],
    )
    return run(x0, x1, x2, bp, vs)


def _tc_kernel(xt_ref, b_ref, vt_ref, out_ref):
    i = pl.program_id(0)

    @pl.when(i == 0)
    def _init():
        out_ref[...] = jnp.zeros_like(out_ref)

    nh = jnp.dot(vt_ref[...], xt_ref[...],
                 preferred_element_type=jnp.float32)
    e0 = jnp.exp(jnp.clip(nh - _C0, -88.0, 88.0)).astype(jnp.bfloat16)
    e1 = jnp.exp(jnp.clip(nh - _C1, -88.0, 88.0)).astype(jnp.bfloat16)
    one = jnp.bfloat16(1.0)
    blocks = []
    for s in range(_S):
        e = e0 if s < _H else e1
        m1 = one + e * jnp.bfloat16(_KCONST[s])
        blocks.append(one / m1)
    sig = jnp.concatenate(blocks, axis=0)
    seg = b_ref[...]
    oh = (seg == lax.broadcasted_iota(jnp.int32, (1, _G), 1)
          ).astype(jnp.bfloat16)
    out_ref[...] += jnp.dot(sig, oh,
                            preferred_element_type=jnp.float32)


def _run_tc(xt, batch2d, vt):
    st = _S * _T
    grid = _N_TC // _TC_CHUNK
    return pl.pallas_call(
        _tc_kernel,
        grid=(grid,),
        in_specs=[
            pl.BlockSpec((3, _TC_CHUNK), lambda i: (0, i)),
            pl.BlockSpec((_TC_CHUNK, 1), lambda i: (i, 0)),
            pl.BlockSpec((_T, 3), lambda i: (0, 0)),
        ],
        out_specs=pl.BlockSpec((st, _G), lambda i: (0, 0)),
        out_shape=jax.ShapeDtypeStruct((st, _G), jnp.float32),
    )(xt, batch2d, vt)


@jax.jit
def kernel(x, batch, v):
    xp = jnp.pad(x, ((0, _NPAD - _N), (0, 0)))
    bp = jnp.pad(batch, (0, _NPAD - _N)).astype(jnp.int32)
    vs = (_SCALE * v).astype(jnp.float32)
    x0, x1, x2 = xp[:, 0], xp[:, 1], xp[:, 2]
    out_sc = _run_sc(x0, x1, x2, bp, vs)
    out = out_sc.sum(axis=0).reshape(_G, _S, _T)
    if _N_TC:
        xt = xp[:_N_TC].T
        batch2d = bp[:_N_TC].reshape(_N_TC, 1)
        out_tc = _run_tc(xt, batch2d, vs.T)
        out = out + out_tc.T.reshape(_G, _S, _T)
    return out
